# Initial kernel scaffold; baseline (speedup 1.0000x reference)
import jax
import jax.numpy as jnp
from jax.experimental import pallas as pl


def kernel(z, pos, edge_index, batch, params):
    raise NotImplementedError("write your pallas kernel here")


# speedup vs baseline: 1.6026x; 1.6026x over previous
import functools

import jax
import jax.numpy as jnp
from jax.experimental import pallas as pl
from jax.experimental.pallas import tpu as pltpu

_INTERPRET = False

NUM_RADIAL = 6
CUTOFF = 5.0
LAMBDA_ATTN = 0.1
TOPK = 8
S_DIM = 16
NEG_HUGE = -3.0e38
BIG_I32 = 2**30


def _sw(x):
    return x * (1.0 / (1.0 + jnp.exp(-x)))


def _rbf8(dist):
    centers = jax.lax.broadcasted_iota(jnp.int32, (1, 8), 1).astype(jnp.float32)
    return jnp.exp(-10.0 * (dist - centers) ** 2)


def _dot(a, b, precision=None):
    return jax.lax.dot_general(a, b, (((1,), (0,)), ((), ())),
                               preferred_element_type=jnp.float32,
                               precision=precision)


def _pad_rows(x, rows):
    if x.shape[0] == rows:
        return x
    return jnp.pad(x, ((0, rows - x.shape[0]),) + ((0, 0),) * (x.ndim - 1))


def _ceil_to(x, m):
    return ((x + m - 1) // m) * m


def _edge_init_body(asj_ref, adi_ref, dist_ref, wrbf_ref, winit_ref,
                    e_ref, rbf_ref, u0_ref):
    rbf = _rbf8(dist_ref[...])
    e = _sw(asj_ref[...] + adi_ref[...] + _dot(rbf, wrbf_ref[...]))
    e_ref[...] = e
    rbf_ref[...] = rbf
    u0_ref[...] = _sw(_dot(e, winit_ref[...]))


def _edge_init(asj, adi, dist, wrbf8, winitv):
    E = asj.shape[0]
    Be = 3200 if E % 3200 == 0 else E
    Ep = _ceil_to(E, Be)
    asj, adi, dist = _pad_rows(asj, Ep), _pad_rows(adi, Ep), _pad_rows(dist, Ep)
    e, rbf, u0 = pl.pallas_call(
        _edge_init_body,
        grid=(Ep // Be,),
        in_specs=[
            pl.BlockSpec((Be, 128), lambda b: (b, 0)),
            pl.BlockSpec((Be, 128), lambda b: (b, 0)),
            pl.BlockSpec((Be, 1), lambda b: (b, 0)),
            pl.BlockSpec((8, 128), lambda b: (0, 0)),
            pl.BlockSpec((128, 128), lambda b: (0, 0)),
        ],
        out_specs=[
            pl.BlockSpec((Be, 128), lambda b: (b, 0)),
            pl.BlockSpec((Be, 8), lambda b: (b, 0)),
            pl.BlockSpec((Be, 128), lambda b: (b, 0)),
        ],
        out_shape=[
            jax.ShapeDtypeStruct((Ep, 128), jnp.float32),
            jax.ShapeDtypeStruct((Ep, 8), jnp.float32),
            jax.ShapeDtypeStruct((Ep, 128), jnp.float32),
        ],
        interpret=_INTERPRET,
    )(asj, adi, dist, wrbf8, winitv)
    return e[:E], rbf[:E], u0[:E]


def _edge_msg_body(e_ref, rbf_ref, wrl_ref, we_ref, u_ref, enew_ref):
    e = e_ref[...]
    rw = _dot(rbf_ref[...], wrl_ref[...])
    u_ref[...] = _sw(e * rw)
    enew_ref[...] = e + _sw(_dot(e, we_ref[...]))


def _edge_msg(e, rbf, wrl8, we):
    E = e.shape[0]
    Be = 3200 if E % 3200 == 0 else E
    Ep = _ceil_to(E, Be)
    e, rbf = _pad_rows(e, Ep), _pad_rows(rbf, Ep)
    u, enew = pl.pallas_call(
        _edge_msg_body,
        grid=(Ep // Be,),
        in_specs=[
            pl.BlockSpec((Be, 128), lambda b: (b, 0)),
            pl.BlockSpec((Be, 8), lambda b: (b, 0)),
            pl.BlockSpec((8, 128), lambda b: (0, 0)),
            pl.BlockSpec((128, 128), lambda b: (0, 0)),
        ],
        out_specs=[
            pl.BlockSpec((Be, 128), lambda b: (b, 0)),
            pl.BlockSpec((Be, 128), lambda b: (b, 0)),
        ],
        out_shape=[
            jax.ShapeDtypeStruct((Ep, 128), jnp.float32),
            jax.ShapeDtypeStruct((Ep, 128), jnp.float32),
        ],
        interpret=_INTERPRET,
    )(e, rbf, wrl8, we)
    return u[:E], enew[:E]


def _edge_msg2_body(e_ref, rbf_ref, esub_ref, wrl_ref, u2_ref):
    rw = _dot(rbf_ref[...], wrl_ref[...])
    u2_ref[...] = _sw(e_ref[...] * rw) * esub_ref[...]


def _edge_msg2(e, rbf, esub, wrl8):
    E = e.shape[0]
    Be = 3200 if E % 3200 == 0 else E
    Ep = _ceil_to(E, Be)
    e, rbf, esub = _pad_rows(e, Ep), _pad_rows(rbf, Ep), _pad_rows(esub, Ep)
    u2 = pl.pallas_call(
        _edge_msg2_body,
        grid=(Ep // Be,),
        in_specs=[
            pl.BlockSpec((Be, 128), lambda b: (b, 0)),
            pl.BlockSpec((Be, 8), lambda b: (b, 0)),
            pl.BlockSpec((Be, 1), lambda b: (b, 0)),
            pl.BlockSpec((8, 128), lambda b: (0, 0)),
        ],
        out_specs=pl.BlockSpec((Be, 128), lambda b: (b, 0)),
        out_shape=jax.ShapeDtypeStruct((Ep, 128), jnp.float32),
        interpret=_INTERPRET,
    )(e, rbf, esub, wrl8)
    return u2[:E]


def _attn_body(q_ref, kt_ref, mask_ref, vals_ref, idx_ref, *, Bm, Bn, Np):
    pid = pl.program_id(0)
    q = q_ref[...]
    row_ids = pid * Bm + jax.lax.broadcasted_iota(jnp.int32, (Bm, Bn), 0)
    cvals, cidx = [], []
    for cb in range(Np // Bn):
        kt = kt_ref[:, pl.ds(cb * Bn, Bn)]
        s = _dot(q, kt, precision=jax.lax.Precision.HIGHEST) * LAMBDA_ATTN
        col_ids = cb * Bn + jax.lax.broadcasted_iota(jnp.int32, (Bm, Bn), 1)
        s = jnp.where(col_ids == row_ids, s - 1e9, s)
        mb = mask_ref[:, pl.ds(cb * Bn, Bn)]
        s = jnp.where(mb > 0.0, s, -1e30)
        for _ in range(TOPK):
            mx = jnp.max(s, axis=1, keepdims=True)
            eq = s == mx
            am = jnp.min(jnp.where(eq, col_ids, BIG_I32), axis=1, keepdims=True)
            cvals.append(mx)
            cidx.append(am)
            s = jnp.where(col_ids == am, NEG_HUGE, s)
    V = jnp.concatenate(cvals, axis=1)
    I = jnp.concatenate(cidx, axis=1)
    ovals, oidx = [], []
    for _ in range(TOPK):
        mx = jnp.max(V, axis=1, keepdims=True)
        eq = V == mx
        chosen = jnp.min(jnp.where(eq, I, BIG_I32), axis=1, keepdims=True)
        ovals.append(mx)
        oidx.append(chosen)
        V = jnp.where(I == chosen, NEG_HUGE, V)
    vals_ref[...] = jnp.concatenate(ovals, axis=1)
    idx_ref[...] = jnp.concatenate(oidx, axis=1)


def _attention_topk(q_all, k_all, maskf):
    N = q_all.shape[0]
    Np = _ceil_to(N, 2048) if N > 256 else _ceil_to(N, 128)
    Bm = min(256, Np)
    Bn = min(2048, Np)
    qp = _pad_rows(q_all, Np)
    ktp = _pad_rows(k_all, Np).T
    mp = _pad_rows(maskf[:, None], Np).T
    body = functools.partial(_attn_body, Bm=Bm, Bn=Bn, Np=Np)
    vals, idx = pl.pallas_call(
        body,
        grid=(Np // Bm,),
        in_specs=[
            pl.BlockSpec((Bm, S_DIM), lambda b: (b, 0)),
            pl.BlockSpec((S_DIM, Np), lambda b: (0, 0)),
            pl.BlockSpec((1, Np), lambda b: (0, 0)),
        ],
        out_specs=[
            pl.BlockSpec((Bm, TOPK), lambda b: (b, 0)),
            pl.BlockSpec((Bm, TOPK), lambda b: (b, 0)),
        ],
        out_shape=[
            jax.ShapeDtypeStruct((Np, TOPK), jnp.float32),
            jax.ShapeDtypeStruct((Np, TOPK), jnp.int32),
        ],
        interpret=_INTERPRET,
    )(qp, ktp, mp)
    return vals[:N], idx[:N]


def _virt_body(vmsj_ref, vmdi_ref, dist_ref, a_ref, valid_ref,
               wrbf_ref, wrl_ref, uv_ref):
    rbf = _rbf8(dist_ref[...])
    ev = a_ref[...] * _sw(vmsj_ref[...] + vmdi_ref[...] + _dot(rbf, wrbf_ref[...]))
    uv_ref[...] = _sw(ev * _dot(rbf, wrl_ref[...])) * valid_ref[...]


def _virt_msg(vmsj, vmdi, dist, a, valid, wrbf8, wrl8):
    M = vmsj.shape[0]
    Bd = 3200 if M % 3200 == 0 else M
    Mp = _ceil_to(M, Bd)
    vmsj, vmdi = _pad_rows(vmsj, Mp), _pad_rows(vmdi, Mp)
    dist, a, valid = _pad_rows(dist, Mp), _pad_rows(a, Mp), _pad_rows(valid, Mp)
    uv = pl.pallas_call(
        _virt_body,
        grid=(Mp // Bd,),
        in_specs=[
            pl.BlockSpec((Bd, 128), lambda b: (b, 0)),
            pl.BlockSpec((Bd, 128), lambda b: (b, 0)),
            pl.BlockSpec((Bd, 1), lambda b: (b, 0)),
            pl.BlockSpec((Bd, 1), lambda b: (b, 0)),
            pl.BlockSpec((Bd, 1), lambda b: (b, 0)),
            pl.BlockSpec((8, 128), lambda b: (0, 0)),
            pl.BlockSpec((8, 128), lambda b: (0, 0)),
        ],
        out_specs=pl.BlockSpec((Bd, 128), lambda b: (b, 0)),
        out_shape=jax.ShapeDtypeStruct((Mp, 128), jnp.float32),
        interpret=_INTERPRET,
    )(vmsj, vmdi, dist, a, valid, wrbf8, wrl8)
    return uv[:M]


def kernel(z, pos, edge_index, batch, params):
    p = params
    f32 = jnp.float32
    N = pos.shape[0]
    j = edge_index[0]
    i = edge_index[1]

    h0 = p["emb_table"][z]
    a_src = h0 @ p["W_src"]
    a_dst = h0 @ p["W_dst"]
    asj = jnp.take(a_src, j, axis=0)
    adi = jnp.take(a_dst, i, axis=0)
    pos_i = jnp.take(pos, i, axis=0)
    pos_j = jnp.take(pos, j, axis=0)
    dvec = pos_i - pos_j
    dist = jnp.sqrt(jnp.sum(dvec * dvec, axis=-1) + 1e-9)[:, None]

    wrbf8 = jnp.zeros((8, 128), f32).at[:NUM_RADIAL].set(p["W_rbf"])
    e, rbf, u0 = _edge_init(asj, adi, dist, wrbf8, p["W_init_v"])
    v = jax.ops.segment_sum(u0, i, num_segments=N)

    ar = jnp.arange(N)
    for lp in p["layers"]:
        wrl8 = jnp.zeros((8, 128), f32).at[:NUM_RADIAL].set(lp["W_rbf_l"])
        u, e = _edge_msg(e, rbf, wrl8, lp["W_e"])
        v_update = jax.ops.segment_sum(u, i, num_segments=N) @ lp["W_msg"]
        v_local = v + v_update
        hs = v_local[:, :S_DIM]
        m = jax.nn.sigmoid(_sw(hs @ lp["W_ms1"] + lp["b_ms1"]) @ lp["W_ms2"]
                           + lp["b_ms2"])[:, 0]
        mask = m > 0.5
        num_master = jnp.sum(mask.astype(jnp.int32))
        esubf = (mask[i] & mask[j]).astype(f32)[:, None]

        q_all = hs @ lp["W_q"]
        k_all = hs @ lp["W_k"]
        vals, nbr = _attention_topk(q_all, k_all, mask.astype(f32))

        kk = jnp.minimum(TOPK, num_master - 1)
        col_valid = jnp.arange(TOPK) < kk
        validf = (mask[:, None] & col_valid[None, :]).astype(f32).reshape(-1, 1)
        A = jax.nn.sigmoid(vals).reshape(-1, 1)
        j_virt = nbr.reshape(-1)

        vm_s = v_local @ p["W_src"]
        vm_d = v_local @ p["W_dst"]
        vmsj = jnp.take(vm_s, j_virt, axis=0)
        vmdi = jnp.repeat(vm_d, TOPK, axis=0)
        dvm = jnp.repeat(pos, TOPK, axis=0) - jnp.take(pos, j_virt, axis=0)
        dist_m = jnp.sqrt(jnp.sum(dvm * dvm, axis=-1) + 1e-9)[:, None]
        uv = _virt_msg(vmsj, vmdi, dist_m, A, validf, wrbf8, wrl8)
        s_virt = uv.reshape(N, TOPK, 128).sum(axis=1)

        u2 = _edge_msg2(e, rbf, esubf, wrl8)
        s2 = jax.ops.segment_sum(u2, i, num_segments=N)
        v_upd_m = (s2 + s_virt) @ lp["W_msg"]

        v_hier = jnp.where(mask[:, None], v_local + v_upd_m, 0.0)
        v_mixed = (1.0 - m[:, None]) * v_local + m[:, None] * v_hier
        v = jnp.where(num_master > 1, v_mixed, v_local)

    pooled = jnp.sum(v, axis=0, keepdims=True)
    out = jax.nn.relu(pooled @ p["W_p1"] + p["b_p1"]) @ p["W_p2"] + p["b_p2"]
    return out


# speedup vs baseline: 1.8034x; 1.1253x over previous
import functools

import jax
import jax.numpy as jnp
from jax import lax
from jax.experimental import pallas as pl
from jax.experimental.pallas import tpu as pltpu
from jax.experimental.pallas import tpu_sc as plsc

_INTERPRET = False

NUM_RADIAL = 6
LAMBDA_ATTN = 0.1
TOPK = 8
S_DIM = 16
NEG_HUGE = -3.0e38
BIG_I32 = 2**30

_CH = 1024
_NW = 32
_BE = 2048
_EALIGN = _NW * _CH


def _sw(x):
    return x * (1.0 / (1.0 + jnp.exp(-x)))


def _rbf8(dist):
    centers = jax.lax.broadcasted_iota(jnp.int32, (1, 8), 1).astype(jnp.float32)
    return jnp.exp(-10.0 * (dist - centers) ** 2)


def _dot(a, b, precision=None):
    return jax.lax.dot_general(a, b, (((1,), (0,)), ((), ())),
                               preferred_element_type=jnp.float32,
                               precision=precision)


def _pad_rows(x, rows):
    if x.shape[0] == rows:
        return x
    return jnp.pad(x, ((0, rows - x.shape[0]),) + ((0, 0),) * (x.ndim - 1))


def _ceil_to(x, m):
    return ((x + m - 1) // m) * m


def _row_valid(pid, Be, Ereal):
    rows = pid * Be + jax.lax.broadcasted_iota(jnp.int32, (Be, 1), 0)
    return (rows < Ereal).astype(jnp.float32)


def _sc_mesh():
    return plsc.VectorSubcoreMesh(core_axis_name="c", subcore_axis_name="s",
                                  num_cores=2, num_subcores=16)


def _zero_vmem(rows_v, nrows):
    zv = jnp.zeros((16,), jnp.float32)

    def zbody(r, c):
        for cc in range(8):
            rows_v[r, pl.ds(cc * 16, 16)] = zv
        return c

    lax.fori_loop(0, nrows, zbody, 0)


def _segsum_sc(u, idx3d, N):
    Ep = u.shape[0]
    C = Ep // (_NW * _CH)
    Np2 = _ceil_to(N, 128)
    RZ = Np2 // 16
    HB = _CH // 4
    NFULL = RZ // HB
    REM = RZ - NFULL * HB

    @functools.partial(
        pl.kernel,
        mesh=_sc_mesh(),
        out_type=jax.ShapeDtypeStruct((2 * Np2, 128), jnp.float32),
        scratch_types=[
            pltpu.VMEM_SHARED((Np2, 128), jnp.float32),
            pltpu.VMEM((8, 128), jnp.int32),
            pltpu.VMEM((HB, 128), jnp.float32),
        ],
    )
    def seg_kernel(u_hbm, idx_hbm, out_hbm, acc_sp, idx_v, rows_v):
        ci = lax.axis_index("c")
        si = lax.axis_index("s")
        wid = si * 2 + ci
        _zero_vmem(rows_v, HB)

        if NFULL:
            def zcopy(kk, c):
                pltpu.sync_copy(rows_v, acc_sp.at[pl.ds(si * RZ + kk * HB, HB)])
                return c
            lax.fori_loop(0, NFULL, zcopy, 0)
        if REM:
            pltpu.sync_copy(rows_v.at[pl.ds(0, REM)],
                            acc_sp.at[pl.ds(si * RZ + NFULL * HB, REM)])
        plsc.subcore_barrier()

        def body(cc, c):
            cid = wid * C + cc
            off = cid * _CH
            pltpu.sync_copy(idx_hbm.at[cid], idx_v)
            for h in range(4):
                pltpu.sync_copy(u_hbm.at[pl.ds(off + h * HB, HB)], rows_v)
                for k in range(2):
                    pltpu.sync_copy(rows_v.at[pl.ds(k * 128, 128)],
                                    acc_sp.at[idx_v.at[h * 2 + k]], add=True)
            return c
        lax.fori_loop(0, C, body, 0)
        plsc.subcore_barrier()

        if NFULL:
            def wcopy(kk, c):
                pltpu.sync_copy(acc_sp.at[pl.ds(si * RZ + kk * HB, HB)],
                                out_hbm.at[pl.ds(ci * Np2 + si * RZ + kk * HB, HB)])
                return c
            lax.fori_loop(0, NFULL, wcopy, 0)
        if REM:
            pltpu.sync_copy(acc_sp.at[pl.ds(si * RZ + NFULL * HB, REM)],
                            out_hbm.at[pl.ds(ci * Np2 + si * RZ + NFULL * HB, REM)])

    part = seg_kernel(u, idx3d)
    return part[:N] + part[Np2:Np2 + N]


def _gather_sc(table, idx3d):
    Ep = idx3d.shape[0] * _CH
    C = Ep // (_NW * _CH)
    HB = _CH // 2

    @functools.partial(
        pl.kernel,
        mesh=_sc_mesh(),
        out_type=jax.ShapeDtypeStruct((Ep, 128), jnp.float32),
        scratch_types=[
            pltpu.VMEM((8, 128), jnp.int32),
            pltpu.VMEM((HB, 128), jnp.float32),
            pltpu.SemaphoreType.DMA,
        ],
    )
    def gather_kernel(tab_hbm, idx_hbm, out_hbm, idx_v, rows_v, sem):
        ci = lax.axis_index("c")
        si = lax.axis_index("s")
        wid = si * 2 + ci

        def body(cc, c):
            cid = wid * C + cc
            off = cid * _CH
            pltpu.sync_copy(idx_hbm.at[cid], idx_v)
            for h in range(2):
                for k in range(4):
                    pltpu.async_copy(tab_hbm.at[idx_v.at[h * 4 + k]],
                                     rows_v.at[pl.ds(k * 128, 128)], sem).wait()
                pltpu.sync_copy(rows_v, out_hbm.at[pl.ds(off + h * HB, HB)])
            return c
        lax.fori_loop(0, C, body, 0)

    return gather_kernel(table, idx3d)


def _edge_init_body(asj_ref, adi_ref, dist_ref, wrbf_ref, winit_ref,
                    e_ref, rbf_ref, u0_ref, *, Be, Ereal):
    rbf = _rbf8(dist_ref[...])
    e = _sw(asj_ref[...] + adi_ref[...] + _dot(rbf, wrbf_ref[...]))
    e_ref[...] = e
    rbf_ref[...] = rbf
    u0_ref[...] = _sw(_dot(e, winit_ref[...])) * _row_valid(pl.program_id(0), Be, Ereal)


def _edge_init(asj, adi, dist, wrbf8, winitv, Ereal):
    Ep = asj.shape[0]
    Be = _BE
    body = functools.partial(_edge_init_body, Be=Be, Ereal=Ereal)
    return pl.pallas_call(
        body,
        grid=(Ep // Be,),
        in_specs=[
            pl.BlockSpec((Be, 128), lambda b: (b, 0)),
            pl.BlockSpec((Be, 128), lambda b: (b, 0)),
            pl.BlockSpec((Be, 1), lambda b: (b, 0)),
            pl.BlockSpec((8, 128), lambda b: (0, 0)),
            pl.BlockSpec((128, 128), lambda b: (0, 0)),
        ],
        out_specs=[
            pl.BlockSpec((Be, 128), lambda b: (b, 0)),
            pl.BlockSpec((Be, 8), lambda b: (b, 0)),
            pl.BlockSpec((Be, 128), lambda b: (b, 0)),
        ],
        out_shape=[
            jax.ShapeDtypeStruct((Ep, 128), jnp.float32),
            jax.ShapeDtypeStruct((Ep, 8), jnp.float32),
            jax.ShapeDtypeStruct((Ep, 128), jnp.float32),
        ],
        interpret=_INTERPRET,
    )(asj, adi, dist, wrbf8, winitv)


def _edge_msg_body(e_ref, rbf_ref, wrl_ref, we_ref, u_ref, enew_ref, *, Be, Ereal):
    e = e_ref[...]
    rw = _dot(rbf_ref[...], wrl_ref[...])
    u_ref[...] = _sw(e * rw) * _row_valid(pl.program_id(0), Be, Ereal)
    enew_ref[...] = e + _sw(_dot(e, we_ref[...]))


def _edge_msg(e, rbf, wrl8, we, Ereal):
    Ep = e.shape[0]
    Be = _BE
    body = functools.partial(_edge_msg_body, Be=Be, Ereal=Ereal)
    return pl.pallas_call(
        body,
        grid=(Ep // Be,),
        in_specs=[
            pl.BlockSpec((Be, 128), lambda b: (b, 0)),
            pl.BlockSpec((Be, 8), lambda b: (b, 0)),
            pl.BlockSpec((8, 128), lambda b: (0, 0)),
            pl.BlockSpec((128, 128), lambda b: (0, 0)),
        ],
        out_specs=[
            pl.BlockSpec((Be, 128), lambda b: (b, 0)),
            pl.BlockSpec((Be, 128), lambda b: (b, 0)),
        ],
        out_shape=[
            jax.ShapeDtypeStruct((Ep, 128), jnp.float32),
            jax.ShapeDtypeStruct((Ep, 128), jnp.float32),
        ],
        interpret=_INTERPRET,
    )(e, rbf, wrl8, we)


def _edge_msg2_body(e_ref, rbf_ref, esub_ref, wrl_ref, u2_ref):
    rw = _dot(rbf_ref[...], wrl_ref[...])
    u2_ref[...] = _sw(e_ref[...] * rw) * esub_ref[...]


def _edge_msg2(e, rbf, esub, wrl8):
    Ep = e.shape[0]
    Be = _BE
    return pl.pallas_call(
        _edge_msg2_body,
        grid=(Ep // Be,),
        in_specs=[
            pl.BlockSpec((Be, 128), lambda b: (b, 0)),
            pl.BlockSpec((Be, 8), lambda b: (b, 0)),
            pl.BlockSpec((Be, 1), lambda b: (b, 0)),
            pl.BlockSpec((8, 128), lambda b: (0, 0)),
        ],
        out_specs=pl.BlockSpec((Be, 128), lambda b: (b, 0)),
        out_shape=jax.ShapeDtypeStruct((Ep, 128), jnp.float32),
        interpret=_INTERPRET,
    )(e, rbf, esub, wrl8)


def _attn_body(q_ref, kt_ref, mask_ref, vals_ref, idx_ref, *, Bm, Bn, Np):
    pid = pl.program_id(0)
    q = q_ref[...]
    row_ids = pid * Bm + jax.lax.broadcasted_iota(jnp.int32, (Bm, Bn), 0)
    cvals, cidx = [], []
    for cb in range(Np // Bn):
        kt = kt_ref[:, pl.ds(cb * Bn, Bn)]
        s = _dot(q, kt, precision=jax.lax.Precision.HIGHEST) * LAMBDA_ATTN
        col_ids = cb * Bn + jax.lax.broadcasted_iota(jnp.int32, (Bm, Bn), 1)
        s = jnp.where(col_ids == row_ids, s - 1e9, s)
        mb = mask_ref[:, pl.ds(cb * Bn, Bn)]
        s = jnp.where(mb > 0.0, s, -1e30)
        for _ in range(TOPK):
            mx = jnp.max(s, axis=1, keepdims=True)
            eq = s == mx
            am = jnp.min(jnp.where(eq, col_ids, BIG_I32), axis=1, keepdims=True)
            cvals.append(mx)
            cidx.append(am)
            s = jnp.where(col_ids == am, NEG_HUGE, s)
    V = jnp.concatenate(cvals, axis=1)
    I = jnp.concatenate(cidx, axis=1)
    ovals, oidx = [], []
    for _ in range(TOPK):
        mx = jnp.max(V, axis=1, keepdims=True)
        eq = V == mx
        chosen = jnp.min(jnp.where(eq, I, BIG_I32), axis=1, keepdims=True)
        ovals.append(mx)
        oidx.append(chosen)
        V = jnp.where(I == chosen, NEG_HUGE, V)
    vals_ref[...] = jnp.concatenate(ovals, axis=1)
    idx_ref[...] = jnp.concatenate(oidx, axis=1)


def _attention_topk(q_all, k_all, maskf):
    N = q_all.shape[0]
    Np = _ceil_to(N, 2048) if N > 256 else _ceil_to(N, 128)
    Bm = min(256, Np)
    Bn = min(2048, Np)
    qp = _pad_rows(q_all, Np)
    ktp = _pad_rows(k_all, Np).T
    mp = _pad_rows(maskf[:, None], Np).T
    body = functools.partial(_attn_body, Bm=Bm, Bn=Bn, Np=Np)
    vals, idx = pl.pallas_call(
        body,
        grid=(Np // Bm,),
        in_specs=[
            pl.BlockSpec((Bm, S_DIM), lambda b: (b, 0)),
            pl.BlockSpec((S_DIM, Np), lambda b: (0, 0)),
            pl.BlockSpec((1, Np), lambda b: (0, 0)),
        ],
        out_specs=[
            pl.BlockSpec((Bm, TOPK), lambda b: (b, 0)),
            pl.BlockSpec((Bm, TOPK), lambda b: (b, 0)),
        ],
        out_shape=[
            jax.ShapeDtypeStruct((Np, TOPK), jnp.float32),
            jax.ShapeDtypeStruct((Np, TOPK), jnp.int32),
        ],
        interpret=_INTERPRET,
    )(qp, ktp, mp)
    return vals[:N], idx[:N]


def _virt_body(vmsj_ref, vmdi_ref, dist_ref, a_ref, valid_ref,
               wrbf_ref, wrl_ref, uv_ref):
    rbf = _rbf8(dist_ref[...])
    ev = a_ref[...] * _sw(vmsj_ref[...] + vmdi_ref[...] + _dot(rbf, wrbf_ref[...]))
    uv_ref[...] = _sw(ev * _dot(rbf, wrl_ref[...])) * valid_ref[...]


def _virt_msg(vmsj, vmdi, dist, a, valid, wrbf8, wrl8):
    Mp = vmsj.shape[0]
    Bd = _BE
    return pl.pallas_call(
        _virt_body,
        grid=(Mp // Bd,),
        in_specs=[
            pl.BlockSpec((Bd, 128), lambda b: (b, 0)),
            pl.BlockSpec((Bd, 128), lambda b: (b, 0)),
            pl.BlockSpec((Bd, 1), lambda b: (b, 0)),
            pl.BlockSpec((Bd, 1), lambda b: (b, 0)),
            pl.BlockSpec((Bd, 1), lambda b: (b, 0)),
            pl.BlockSpec((8, 128), lambda b: (0, 0)),
            pl.BlockSpec((8, 128), lambda b: (0, 0)),
        ],
        out_specs=pl.BlockSpec((Bd, 128), lambda b: (b, 0)),
        out_shape=jax.ShapeDtypeStruct((Mp, 128), jnp.float32),
        interpret=_INTERPRET,
    )(vmsj, vmdi, dist, a, valid, wrbf8, wrl8)


_SEGSUM = _segsum_sc
_GATHER = _gather_sc


def kernel(z, pos, edge_index, batch, params):
    p = params
    f32 = jnp.float32
    N = pos.shape[0]
    E = edge_index.shape[1]
    Ep = _ceil_to(E, _EALIGN)
    j = edge_index[0]
    i = edge_index[1]
    jp_ = jnp.pad(j, (0, Ep - E)).astype(jnp.int32)
    ip_ = jnp.pad(i, (0, Ep - E)).astype(jnp.int32)
    j2d = jp_.reshape(Ep // _CH, 8, 128)
    i2d = ip_.reshape(Ep // _CH, 8, 128)

    h0 = p["emb_table"][z]
    a_src = h0 @ p["W_src"]
    a_dst = h0 @ p["W_dst"]
    asj = _GATHER(a_src, j2d)
    adi = _GATHER(a_dst, i2d)
    dvec = jnp.take(pos, i, axis=0) - jnp.take(pos, j, axis=0)
    dist = _pad_rows(jnp.sqrt(jnp.sum(dvec * dvec, axis=-1) + 1e-9)[:, None], Ep)

    wrbf8 = jnp.zeros((8, 128), f32).at[:NUM_RADIAL].set(p["W_rbf"])
    e, rbf, u0 = _edge_init(asj, adi, dist, wrbf8, p["W_init_v"], E)
    v = _SEGSUM(u0, i2d, N)

    M = N * TOPK
    Mp = _ceil_to(M, _EALIGN)
    iv_ = jnp.pad(jnp.repeat(jnp.arange(N, dtype=jnp.int32), TOPK), (0, Mp - M))
    iv2d = iv_.reshape(Mp // _CH, 8, 128)

    for lp in p["layers"]:
        wrl8 = jnp.zeros((8, 128), f32).at[:NUM_RADIAL].set(lp["W_rbf_l"])
        u, e = _edge_msg(e, rbf, wrl8, lp["W_e"], E)
        v_update = _SEGSUM(u, i2d, N) @ lp["W_msg"]
        v_local = v + v_update
        hs = v_local[:, :S_DIM]
        m = jax.nn.sigmoid(_sw(hs @ lp["W_ms1"] + lp["b_ms1"]) @ lp["W_ms2"]
                           + lp["b_ms2"])[:, 0]
        mask = m > 0.5
        num_master = jnp.sum(mask.astype(jnp.int32))
        esubf = (mask[ip_] & mask[jp_]).astype(f32)[:, None]
        esubf = esubf * (jnp.arange(Ep, dtype=jnp.int32) < E).astype(f32)[:, None]

        q_all = hs @ lp["W_q"]
        k_all = hs @ lp["W_k"]
        vals, nbr = _attention_topk(q_all, k_all, mask.astype(f32))

        kk = jnp.minimum(TOPK, num_master - 1)
        col_valid = jnp.arange(TOPK) < kk
        validf = _pad_rows((mask[:, None] & col_valid[None, :]).astype(f32).reshape(-1, 1), Mp)
        A = _pad_rows(jax.nn.sigmoid(vals).reshape(-1, 1), Mp)
        j_virt = nbr.reshape(-1)
        jv_ = jnp.pad(j_virt, (0, Mp - M)).astype(jnp.int32)
        jv2d = jv_.reshape(Mp // _CH, 8, 128)

        vm_s = v_local @ p["W_src"]
        vm_d = v_local @ p["W_dst"]
        vmsj = _GATHER(vm_s, jv2d)
        vmdi = _GATHER(vm_d, iv2d)
        dvm = jnp.repeat(pos, TOPK, axis=0) - jnp.take(pos, j_virt, axis=0)
        dist_m = _pad_rows(jnp.sqrt(jnp.sum(dvm * dvm, axis=-1) + 1e-9)[:, None], Mp)
        uv = _virt_msg(vmsj, vmdi, dist_m, A, validf, wrbf8, wrl8)
        s_virt = uv.reshape(Mp // TOPK, TOPK, 128).sum(axis=1)[:N]

        u2 = _edge_msg2(e, rbf, esubf, wrl8)
        s2 = _SEGSUM(u2, i2d, N)
        v_upd_m = (s2 + s_virt) @ lp["W_msg"]

        v_hier = jnp.where(mask[:, None], v_local + v_upd_m, 0.0)
        v_mixed = (1.0 - m[:, None]) * v_local + m[:, None] * v_hier
        v = jnp.where(num_master > 1, v_mixed, v_local)

    pooled = jnp.sum(v, axis=0, keepdims=True)
    out = jax.nn.relu(pooled @ p["W_p1"] + p["b_p1"]) @ p["W_p2"] + p["b_p2"]
    return out


# speedup vs baseline: 1.8055x; 1.0012x over previous
import functools

import jax
import jax.numpy as jnp
from jax import lax
from jax.experimental import pallas as pl
from jax.experimental.pallas import tpu as pltpu
from jax.experimental.pallas import tpu_sc as plsc

_INTERPRET = False

NUM_RADIAL = 6
LAMBDA_ATTN = 0.1
TOPK = 8
S_DIM = 16
NEG_HUGE = -3.0e38
BIG_I32 = 2**30

_CH = 1024
_NW = 32
_BE = 2048
_EALIGN = _NW * _CH


def _sw(x):
    return x * (1.0 / (1.0 + jnp.exp(-x)))


def _rbf8(dist):
    centers = jax.lax.broadcasted_iota(jnp.int32, (1, 8), 1).astype(jnp.float32)
    return jnp.exp(-10.0 * (dist - centers) ** 2)


def _dot(a, b, precision=None):
    return jax.lax.dot_general(a, b, (((1,), (0,)), ((), ())),
                               preferred_element_type=jnp.float32,
                               precision=precision)


def _pad_rows(x, rows):
    if x.shape[0] == rows:
        return x
    return jnp.pad(x, ((0, rows - x.shape[0]),) + ((0, 0),) * (x.ndim - 1))


def _ceil_to(x, m):
    return ((x + m - 1) // m) * m


def _row_valid(pid, Be, Ereal):
    rows = pid * Be + jax.lax.broadcasted_iota(jnp.int32, (Be, 1), 0)
    return (rows < Ereal).astype(jnp.float32)


def _sc_mesh():
    return plsc.VectorSubcoreMesh(core_axis_name="c", subcore_axis_name="s",
                                  num_cores=2, num_subcores=16)


def _zero_vmem(rows_v, nrows):
    zv = jnp.zeros((16,), jnp.float32)

    def zbody(r, c):
        for cc in range(8):
            rows_v[r, pl.ds(cc * 16, 16)] = zv
        return c

    lax.fori_loop(0, nrows, zbody, 0)


def _segsum_sc(u, idx3d, N):
    Ep = u.shape[0]
    C = Ep // (_NW * _CH)
    Np2 = _ceil_to(N, 128)
    RZ = Np2 // 16

    @functools.partial(
        pl.kernel,
        mesh=_sc_mesh(),
        out_type=jax.ShapeDtypeStruct((2 * Np2, 128), jnp.float32),
        scratch_types=[
            pltpu.VMEM_SHARED((Np2, 128), jnp.float32),
            pltpu.VMEM((8, 128), jnp.int32),
            pltpu.VMEM((2, 128, 128), jnp.float32),
            pltpu.SemaphoreType.DMA,
            pltpu.SemaphoreType.DMA,
        ],
    )
    def seg_kernel(u_hbm, idx_hbm, out_hbm, acc_sp, idx_v, rows_v, sem_a, sem_b):
        ci = lax.axis_index("c")
        si = lax.axis_index("s")
        wid = si * 2 + ci
        _zero_vmem(rows_v.at[0], 128)

        nzfull = RZ // 128
        def zcopy(kk, c):
            pltpu.sync_copy(rows_v.at[0], acc_sp.at[pl.ds(si * RZ + kk * 128, 128)])
            return c
        lax.fori_loop(0, nzfull, zcopy, 0)
        zrem = RZ - nzfull * 128
        if zrem:
            pltpu.sync_copy(rows_v.at[0].at[pl.ds(0, zrem)],
                            acc_sp.at[pl.ds(si * RZ + nzfull * 128, zrem)])
        plsc.subcore_barrier()

        sems = (sem_a, sem_b)

        def body(cc, c):
            cid = wid * C + cc
            off = cid * _CH
            pltpu.sync_copy(idx_hbm.at[cid], idx_v)
            cp = pltpu.async_copy(u_hbm.at[pl.ds(off, 128)], rows_v.at[0], sem_a)
            for h in range(8):
                cur = h % 2
                if h < 7:
                    cpn = pltpu.async_copy(
                        u_hbm.at[pl.ds(off + (h + 1) * 128, 128)],
                        rows_v.at[1 - cur], sems[1 - cur])
                cp.wait()
                pltpu.sync_copy(rows_v.at[cur], acc_sp.at[idx_v.at[h]], add=True)
                if h < 7:
                    cp = cpn
            return c
        lax.fori_loop(0, C, body, 0)
        plsc.subcore_barrier()

        nzfull2 = RZ // 128
        def wcopy(kk, c):
            pltpu.sync_copy(acc_sp.at[pl.ds(si * RZ + kk * 128, 128)],
                            out_hbm.at[pl.ds(ci * Np2 + si * RZ + kk * 128, 128)])
            return c
        lax.fori_loop(0, nzfull2, wcopy, 0)
        if zrem:
            pltpu.sync_copy(acc_sp.at[pl.ds(si * RZ + nzfull2 * 128, zrem)],
                            out_hbm.at[pl.ds(ci * Np2 + si * RZ + nzfull2 * 128, zrem)])

    part = seg_kernel(u, idx3d)
    return part[:N] + part[Np2:Np2 + N]


def _gather_sc(table, idx3d):
    Ep = idx3d.shape[0] * _CH
    C = Ep // (_NW * _CH)
    @functools.partial(
        pl.kernel,
        mesh=_sc_mesh(),
        out_type=jax.ShapeDtypeStruct((Ep, 128), jnp.float32),
        scratch_types=[
            pltpu.VMEM((8, 128), jnp.int32),
            pltpu.VMEM((2, 128, 128), jnp.float32),
            pltpu.SemaphoreType.DMA,
            pltpu.SemaphoreType.DMA,
        ],
    )
    def gather_kernel(tab_hbm, idx_hbm, out_hbm, idx_v, rows_v, sem_a, sem_b):
        ci = lax.axis_index("c")
        si = lax.axis_index("s")
        wid = si * 2 + ci
        sems = (sem_a, sem_b)

        def body(cc, c):
            cid = wid * C + cc
            off = cid * _CH
            pltpu.sync_copy(idx_hbm.at[cid], idx_v)
            cp = pltpu.async_copy(tab_hbm.at[idx_v.at[0]], rows_v.at[0], sem_a)
            for h in range(8):
                cur = h % 2
                if h < 7:
                    cpn = pltpu.async_copy(tab_hbm.at[idx_v.at[h + 1]],
                                           rows_v.at[1 - cur], sems[1 - cur])
                cp.wait()
                pltpu.sync_copy(rows_v.at[cur],
                                out_hbm.at[pl.ds(off + h * 128, 128)])
                if h < 7:
                    cp = cpn
            return c
        lax.fori_loop(0, C, body, 0)

    return gather_kernel(table, idx3d)


def _edge_init_body(asj_ref, adi_ref, dist_ref, wrbf_ref, winit_ref,
                    e_ref, rbf_ref, u0_ref, *, Be, Ereal):
    rbf = _rbf8(dist_ref[...])
    e = _sw(asj_ref[...] + adi_ref[...] + _dot(rbf, wrbf_ref[...]))
    e_ref[...] = e
    rbf_ref[...] = rbf
    u0_ref[...] = _sw(_dot(e, winit_ref[...])) * _row_valid(pl.program_id(0), Be, Ereal)


def _edge_init(asj, adi, dist, wrbf8, winitv, Ereal):
    Ep = asj.shape[0]
    Be = _BE
    body = functools.partial(_edge_init_body, Be=Be, Ereal=Ereal)
    return pl.pallas_call(
        body,
        grid=(Ep // Be,),
        in_specs=[
            pl.BlockSpec((Be, 128), lambda b: (b, 0)),
            pl.BlockSpec((Be, 128), lambda b: (b, 0)),
            pl.BlockSpec((Be, 1), lambda b: (b, 0)),
            pl.BlockSpec((8, 128), lambda b: (0, 0)),
            pl.BlockSpec((128, 128), lambda b: (0, 0)),
        ],
        out_specs=[
            pl.BlockSpec((Be, 128), lambda b: (b, 0)),
            pl.BlockSpec((Be, 8), lambda b: (b, 0)),
            pl.BlockSpec((Be, 128), lambda b: (b, 0)),
        ],
        out_shape=[
            jax.ShapeDtypeStruct((Ep, 128), jnp.float32),
            jax.ShapeDtypeStruct((Ep, 8), jnp.float32),
            jax.ShapeDtypeStruct((Ep, 128), jnp.float32),
        ],
        interpret=_INTERPRET,
    )(asj, adi, dist, wrbf8, winitv)


def _edge_msg_body(e_ref, rbf_ref, wrl_ref, we_ref, u_ref, enew_ref, *, Be, Ereal):
    e = e_ref[...]
    rw = _dot(rbf_ref[...], wrl_ref[...])
    u_ref[...] = _sw(e * rw) * _row_valid(pl.program_id(0), Be, Ereal)
    enew_ref[...] = e + _sw(_dot(e, we_ref[...]))


def _edge_msg(e, rbf, wrl8, we, Ereal):
    Ep = e.shape[0]
    Be = _BE
    body = functools.partial(_edge_msg_body, Be=Be, Ereal=Ereal)
    return pl.pallas_call(
        body,
        grid=(Ep // Be,),
        in_specs=[
            pl.BlockSpec((Be, 128), lambda b: (b, 0)),
            pl.BlockSpec((Be, 8), lambda b: (b, 0)),
            pl.BlockSpec((8, 128), lambda b: (0, 0)),
            pl.BlockSpec((128, 128), lambda b: (0, 0)),
        ],
        out_specs=[
            pl.BlockSpec((Be, 128), lambda b: (b, 0)),
            pl.BlockSpec((Be, 128), lambda b: (b, 0)),
        ],
        out_shape=[
            jax.ShapeDtypeStruct((Ep, 128), jnp.float32),
            jax.ShapeDtypeStruct((Ep, 128), jnp.float32),
        ],
        interpret=_INTERPRET,
    )(e, rbf, wrl8, we)


def _edge_msg2_body(e_ref, rbf_ref, esub_ref, wrl_ref, u2_ref):
    rw = _dot(rbf_ref[...], wrl_ref[...])
    u2_ref[...] = _sw(e_ref[...] * rw) * esub_ref[...]


def _edge_msg2(e, rbf, esub, wrl8):
    Ep = e.shape[0]
    Be = _BE
    return pl.pallas_call(
        _edge_msg2_body,
        grid=(Ep // Be,),
        in_specs=[
            pl.BlockSpec((Be, 128), lambda b: (b, 0)),
            pl.BlockSpec((Be, 8), lambda b: (b, 0)),
            pl.BlockSpec((Be, 1), lambda b: (b, 0)),
            pl.BlockSpec((8, 128), lambda b: (0, 0)),
        ],
        out_specs=pl.BlockSpec((Be, 128), lambda b: (b, 0)),
        out_shape=jax.ShapeDtypeStruct((Ep, 128), jnp.float32),
        interpret=_INTERPRET,
    )(e, rbf, esub, wrl8)


def _attn_body(q_ref, kt_ref, mask_ref, vals_ref, idx_ref, *, Bm, Bn, Np):
    pid = pl.program_id(0)
    q = q_ref[...]
    row_ids = pid * Bm + jax.lax.broadcasted_iota(jnp.int32, (Bm, Bn), 0)
    cvals, cidx = [], []
    for cb in range(Np // Bn):
        kt = kt_ref[:, pl.ds(cb * Bn, Bn)]
        s = _dot(q, kt, precision=jax.lax.Precision.HIGHEST) * LAMBDA_ATTN
        col_ids = cb * Bn + jax.lax.broadcasted_iota(jnp.int32, (Bm, Bn), 1)
        s = jnp.where(col_ids == row_ids, s - 1e9, s)
        mb = mask_ref[:, pl.ds(cb * Bn, Bn)]
        s = jnp.where(mb > 0.0, s, -1e30)
        for _ in range(TOPK):
            mx = jnp.max(s, axis=1, keepdims=True)
            eq = s == mx
            am = jnp.min(jnp.where(eq, col_ids, BIG_I32), axis=1, keepdims=True)
            cvals.append(mx)
            cidx.append(am)
            s = jnp.where(col_ids == am, NEG_HUGE, s)
    V = jnp.concatenate(cvals, axis=1)
    I = jnp.concatenate(cidx, axis=1)
    ovals, oidx = [], []
    for _ in range(TOPK):
        mx = jnp.max(V, axis=1, keepdims=True)
        eq = V == mx
        chosen = jnp.min(jnp.where(eq, I, BIG_I32), axis=1, keepdims=True)
        ovals.append(mx)
        oidx.append(chosen)
        V = jnp.where(I == chosen, NEG_HUGE, V)
    vals_ref[...] = jnp.concatenate(ovals, axis=1)
    idx_ref[...] = jnp.concatenate(oidx, axis=1)


def _attention_topk(q_all, k_all, maskf):
    N = q_all.shape[0]
    Np = _ceil_to(N, 2048) if N > 256 else _ceil_to(N, 128)
    Bm = min(256, Np)
    Bn = min(2048, Np)
    qp = _pad_rows(q_all, Np)
    ktp = _pad_rows(k_all, Np).T
    mp = _pad_rows(maskf[:, None], Np).T
    body = functools.partial(_attn_body, Bm=Bm, Bn=Bn, Np=Np)
    vals, idx = pl.pallas_call(
        body,
        grid=(Np // Bm,),
        in_specs=[
            pl.BlockSpec((Bm, S_DIM), lambda b: (b, 0)),
            pl.BlockSpec((S_DIM, Np), lambda b: (0, 0)),
            pl.BlockSpec((1, Np), lambda b: (0, 0)),
        ],
        out_specs=[
            pl.BlockSpec((Bm, TOPK), lambda b: (b, 0)),
            pl.BlockSpec((Bm, TOPK), lambda b: (b, 0)),
        ],
        out_shape=[
            jax.ShapeDtypeStruct((Np, TOPK), jnp.float32),
            jax.ShapeDtypeStruct((Np, TOPK), jnp.int32),
        ],
        interpret=_INTERPRET,
    )(qp, ktp, mp)
    return vals[:N], idx[:N]


def _virt_body(vmsj_ref, vmdi_ref, dist_ref, a_ref, valid_ref,
               wrbf_ref, wrl_ref, uv_ref):
    rbf = _rbf8(dist_ref[...])
    ev = a_ref[...] * _sw(vmsj_ref[...] + vmdi_ref[...] + _dot(rbf, wrbf_ref[...]))
    uv_ref[...] = _sw(ev * _dot(rbf, wrl_ref[...])) * valid_ref[...]


def _virt_msg(vmsj, vmdi, dist, a, valid, wrbf8, wrl8):
    Mp = vmsj.shape[0]
    Bd = _BE
    return pl.pallas_call(
        _virt_body,
        grid=(Mp // Bd,),
        in_specs=[
            pl.BlockSpec((Bd, 128), lambda b: (b, 0)),
            pl.BlockSpec((Bd, 128), lambda b: (b, 0)),
            pl.BlockSpec((Bd, 1), lambda b: (b, 0)),
            pl.BlockSpec((Bd, 1), lambda b: (b, 0)),
            pl.BlockSpec((Bd, 1), lambda b: (b, 0)),
            pl.BlockSpec((8, 128), lambda b: (0, 0)),
            pl.BlockSpec((8, 128), lambda b: (0, 0)),
        ],
        out_specs=pl.BlockSpec((Bd, 128), lambda b: (b, 0)),
        out_shape=jax.ShapeDtypeStruct((Mp, 128), jnp.float32),
        interpret=_INTERPRET,
    )(vmsj, vmdi, dist, a, valid, wrbf8, wrl8)


_SEGSUM = _segsum_sc
_GATHER = _gather_sc


def kernel(z, pos, edge_index, batch, params):
    p = params
    f32 = jnp.float32
    N = pos.shape[0]
    E = edge_index.shape[1]
    Ep = _ceil_to(E, _EALIGN)
    j = edge_index[0]
    i = edge_index[1]
    jp_ = jnp.pad(j, (0, Ep - E)).astype(jnp.int32)
    ip_ = jnp.pad(i, (0, Ep - E)).astype(jnp.int32)
    j2d = jp_.reshape(Ep // _CH, 8, 128)
    i2d = ip_.reshape(Ep // _CH, 8, 128)

    h0 = p["emb_table"][z]
    a_src = h0 @ p["W_src"]
    a_dst = h0 @ p["W_dst"]
    asj = _GATHER(a_src, j2d)
    adi = _GATHER(a_dst, i2d)
    dvec = jnp.take(pos, i, axis=0) - jnp.take(pos, j, axis=0)
    dist = _pad_rows(jnp.sqrt(jnp.sum(dvec * dvec, axis=-1) + 1e-9)[:, None], Ep)

    wrbf8 = jnp.zeros((8, 128), f32).at[:NUM_RADIAL].set(p["W_rbf"])
    e, rbf, u0 = _edge_init(asj, adi, dist, wrbf8, p["W_init_v"], E)
    v = _SEGSUM(u0, i2d, N)

    M = N * TOPK
    Mp = _ceil_to(M, _EALIGN)
    iv_ = jnp.pad(jnp.repeat(jnp.arange(N, dtype=jnp.int32), TOPK), (0, Mp - M))
    iv2d = iv_.reshape(Mp // _CH, 8, 128)

    for lp in p["layers"]:
        wrl8 = jnp.zeros((8, 128), f32).at[:NUM_RADIAL].set(lp["W_rbf_l"])
        u, e = _edge_msg(e, rbf, wrl8, lp["W_e"], E)
        v_update = _SEGSUM(u, i2d, N) @ lp["W_msg"]
        v_local = v + v_update
        hs = v_local[:, :S_DIM]
        m = jax.nn.sigmoid(_sw(hs @ lp["W_ms1"] + lp["b_ms1"]) @ lp["W_ms2"]
                           + lp["b_ms2"])[:, 0]
        mask = m > 0.5
        num_master = jnp.sum(mask.astype(jnp.int32))
        esubf = (mask[ip_] & mask[jp_]).astype(f32)[:, None]
        esubf = esubf * (jnp.arange(Ep, dtype=jnp.int32) < E).astype(f32)[:, None]

        q_all = hs @ lp["W_q"]
        k_all = hs @ lp["W_k"]
        vals, nbr = _attention_topk(q_all, k_all, mask.astype(f32))

        kk = jnp.minimum(TOPK, num_master - 1)
        col_valid = jnp.arange(TOPK) < kk
        validf = _pad_rows((mask[:, None] & col_valid[None, :]).astype(f32).reshape(-1, 1), Mp)
        A = _pad_rows(jax.nn.sigmoid(vals).reshape(-1, 1), Mp)
        j_virt = nbr.reshape(-1)
        jv_ = jnp.pad(j_virt, (0, Mp - M)).astype(jnp.int32)
        jv2d = jv_.reshape(Mp // _CH, 8, 128)

        vm_s = v_local @ p["W_src"]
        vm_d = v_local @ p["W_dst"]
        vmsj = _GATHER(vm_s, jv2d)
        vmdi = _GATHER(vm_d, iv2d)
        dvm = jnp.repeat(pos, TOPK, axis=0) - jnp.take(pos, j_virt, axis=0)
        dist_m = _pad_rows(jnp.sqrt(jnp.sum(dvm * dvm, axis=-1) + 1e-9)[:, None], Mp)
        uv = _virt_msg(vmsj, vmdi, dist_m, A, validf, wrbf8, wrl8)
        s_virt = uv.reshape(Mp // TOPK, TOPK, 128).sum(axis=1)[:N]

        u2 = _edge_msg2(e, rbf, esubf, wrl8)
        s2 = _SEGSUM(u2, i2d, N)
        v_upd_m = (s2 + s_virt) @ lp["W_msg"]

        v_hier = jnp.where(mask[:, None], v_local + v_upd_m, 0.0)
        v_mixed = (1.0 - m[:, None]) * v_local + m[:, None] * v_hier
        v = jnp.where(num_master > 1, v_mixed, v_local)

    pooled = jnp.sum(v, axis=0, keepdims=True)
    out = jax.nn.relu(pooled @ p["W_p1"] + p["b_p1"]) @ p["W_p2"] + p["b_p2"]
    return out


# speedup vs baseline: 2.8006x; 1.5511x over previous
import functools

import jax
import jax.numpy as jnp
from jax import lax
from jax.experimental import pallas as pl
from jax.experimental.pallas import tpu as pltpu
from jax.experimental.pallas import tpu_sc as plsc

_INTERPRET = False

NUM_RADIAL = 6
LAMBDA_ATTN = 0.1
TOPK = 8
S_DIM = 16
NEG_HUGE = -3.0e38
BIG_I32 = 2**30

_CH = 1024
_NW = 32
_BE = 2048
_EALIGN = _NW * _CH


def _sw(x):
    return x * (1.0 / (1.0 + jnp.exp(-x)))


def _rbf8(dist):
    centers = jax.lax.broadcasted_iota(jnp.int32, (1, 8), 1).astype(jnp.float32)
    return jnp.exp(-10.0 * (dist - centers) ** 2)


def _dot(a, b, precision=None):
    return jax.lax.dot_general(a, b, (((1,), (0,)), ((), ())),
                               preferred_element_type=jnp.float32,
                               precision=precision)


def _pad_rows(x, rows):
    if x.shape[0] == rows:
        return x
    return jnp.pad(x, ((0, rows - x.shape[0]),) + ((0, 0),) * (x.ndim - 1))


def _ceil_to(x, m):
    return ((x + m - 1) // m) * m


def _row_valid(pid, Be, Ereal):
    rows = pid * Be + jax.lax.broadcasted_iota(jnp.int32, (Be, 1), 0)
    return (rows < Ereal).astype(jnp.float32)


def _sc_mesh():
    return plsc.VectorSubcoreMesh(core_axis_name="c", subcore_axis_name="s",
                                  num_cores=2, num_subcores=16)


def _zero_vmem(rows_v, nrows):
    zv = jnp.zeros((16,), jnp.float32)

    def zbody(r, c):
        for cc in range(8):
            rows_v[r, pl.ds(cc * 16, 16)] = zv
        return c

    lax.fori_loop(0, nrows, zbody, 0)


def _segsum_sc(u, idx3d, N):
    Ep = u.shape[0]
    C = Ep // (_NW * _CH)
    Np2 = _ceil_to(N, 128)
    RZ = Np2 // 16

    @functools.partial(
        pl.kernel,
        mesh=_sc_mesh(),
        out_type=jax.ShapeDtypeStruct((2 * Np2, 128), jnp.float32),
        scratch_types=[
            pltpu.VMEM_SHARED((Np2, 128), jnp.float32),
            pltpu.VMEM((8, 128), jnp.int32),
            pltpu.VMEM((2, 128, 128), jnp.float32),
            pltpu.SemaphoreType.DMA,
            pltpu.SemaphoreType.DMA,
        ],
    )
    def seg_kernel(u_hbm, idx_hbm, out_hbm, acc_sp, idx_v, rows_v, sem_a, sem_b):
        ci = lax.axis_index("c")
        si = lax.axis_index("s")
        wid = si * 2 + ci
        _zero_vmem(rows_v.at[0], 128)

        nzfull = RZ // 128
        def zcopy(kk, c):
            pltpu.sync_copy(rows_v.at[0], acc_sp.at[pl.ds(si * RZ + kk * 128, 128)])
            return c
        lax.fori_loop(0, nzfull, zcopy, 0)
        zrem = RZ - nzfull * 128
        if zrem:
            pltpu.sync_copy(rows_v.at[0].at[pl.ds(0, zrem)],
                            acc_sp.at[pl.ds(si * RZ + nzfull * 128, zrem)])
        plsc.subcore_barrier()

        sems = (sem_a, sem_b)

        def body(cc, c):
            cid = wid * C + cc
            off = cid * _CH
            pltpu.sync_copy(idx_hbm.at[cid], idx_v)
            cp = pltpu.async_copy(u_hbm.at[pl.ds(off, 128)], rows_v.at[0], sem_a)
            for h in range(8):
                cur = h % 2
                if h < 7:
                    cpn = pltpu.async_copy(
                        u_hbm.at[pl.ds(off + (h + 1) * 128, 128)],
                        rows_v.at[1 - cur], sems[1 - cur])
                cp.wait()
                pltpu.sync_copy(rows_v.at[cur], acc_sp.at[idx_v.at[h]], add=True)
                if h < 7:
                    cp = cpn
            return c
        lax.fori_loop(0, C, body, 0)
        plsc.subcore_barrier()

        nzfull2 = RZ // 128
        def wcopy(kk, c):
            pltpu.sync_copy(acc_sp.at[pl.ds(si * RZ + kk * 128, 128)],
                            out_hbm.at[pl.ds(ci * Np2 + si * RZ + kk * 128, 128)])
            return c
        lax.fori_loop(0, nzfull2, wcopy, 0)
        if zrem:
            pltpu.sync_copy(acc_sp.at[pl.ds(si * RZ + nzfull2 * 128, zrem)],
                            out_hbm.at[pl.ds(ci * Np2 + si * RZ + nzfull2 * 128, zrem)])

    part = seg_kernel(u, idx3d)
    return part[:N] + part[Np2:Np2 + N]


def _gather_sc(table, idx3d):
    W = table.shape[1]
    Ep = idx3d.shape[0] * _CH
    C = Ep // (_NW * _CH)
    @functools.partial(
        pl.kernel,
        mesh=_sc_mesh(),
        out_type=jax.ShapeDtypeStruct((Ep, W), jnp.float32),
        scratch_types=[
            pltpu.VMEM((8, 128), jnp.int32),
            pltpu.VMEM((2, 128, W), jnp.float32),
            pltpu.SemaphoreType.DMA,
            pltpu.SemaphoreType.DMA,
        ],
    )
    def gather_kernel(tab_hbm, idx_hbm, out_hbm, idx_v, rows_v, sem_a, sem_b):
        ci = lax.axis_index("c")
        si = lax.axis_index("s")
        wid = si * 2 + ci
        sems = (sem_a, sem_b)

        def body(cc, c):
            cid = wid * C + cc
            off = cid * _CH
            pltpu.sync_copy(idx_hbm.at[cid], idx_v)
            cp = pltpu.async_copy(tab_hbm.at[idx_v.at[0]], rows_v.at[0], sem_a)
            for h in range(8):
                cur = h % 2
                if h < 7:
                    cpn = pltpu.async_copy(tab_hbm.at[idx_v.at[h + 1]],
                                           rows_v.at[1 - cur], sems[1 - cur])
                cp.wait()
                pltpu.sync_copy(rows_v.at[cur],
                                out_hbm.at[pl.ds(off + h * 128, 128)])
                if h < 7:
                    cp = cpn
            return c
        lax.fori_loop(0, C, body, 0)

    return gather_kernel(table, idx3d)


def _edge_init_body(asj_ref, adi_ref, posi_ref, posj_ref, wrbf_ref, winit_ref,
                    e_ref, rbf_ref, u0_ref, *, Be, Ereal):
    d = posi_ref[...] - posj_ref[...]
    dist = jnp.sqrt(jnp.sum(d * d, axis=1, keepdims=True) + 1e-9)
    rbf = _rbf8(dist)
    e = _sw(asj_ref[...] + adi_ref[...] + _dot(rbf, wrbf_ref[...]))
    e_ref[...] = e
    rbf_ref[...] = rbf
    u0_ref[...] = _sw(_dot(e, winit_ref[...])) * _row_valid(pl.program_id(0), Be, Ereal)


def _edge_init(asj, adi, posi, posj, wrbf8, winitv, Ereal):
    Ep = asj.shape[0]
    Be = _BE
    body = functools.partial(_edge_init_body, Be=Be, Ereal=Ereal)
    return pl.pallas_call(
        body,
        grid=(Ep // Be,),
        in_specs=[
            pl.BlockSpec((Be, 128), lambda b: (b, 0)),
            pl.BlockSpec((Be, 128), lambda b: (b, 0)),
            pl.BlockSpec((Be, 128), lambda b: (b, 0)),
            pl.BlockSpec((Be, 128), lambda b: (b, 0)),
            pl.BlockSpec((8, 128), lambda b: (0, 0)),
            pl.BlockSpec((128, 128), lambda b: (0, 0)),
        ],
        out_specs=[
            pl.BlockSpec((Be, 128), lambda b: (b, 0)),
            pl.BlockSpec((Be, 8), lambda b: (b, 0)),
            pl.BlockSpec((Be, 128), lambda b: (b, 0)),
        ],
        out_shape=[
            jax.ShapeDtypeStruct((Ep, 128), jnp.float32),
            jax.ShapeDtypeStruct((Ep, 8), jnp.float32),
            jax.ShapeDtypeStruct((Ep, 128), jnp.float32),
        ],
        interpret=_INTERPRET,
    )(asj, adi, posi, posj, wrbf8, winitv)


def _edge_msg_body(e_ref, rbf_ref, wrl_ref, we_ref, u_ref, enew_ref, *, Be, Ereal):
    e = e_ref[...]
    rw = _dot(rbf_ref[...], wrl_ref[...])
    u_ref[...] = _sw(e * rw) * _row_valid(pl.program_id(0), Be, Ereal)
    enew_ref[...] = e + _sw(_dot(e, we_ref[...]))


def _edge_msg(e, rbf, wrl8, we, Ereal):
    Ep = e.shape[0]
    Be = _BE
    body = functools.partial(_edge_msg_body, Be=Be, Ereal=Ereal)
    return pl.pallas_call(
        body,
        grid=(Ep // Be,),
        in_specs=[
            pl.BlockSpec((Be, 128), lambda b: (b, 0)),
            pl.BlockSpec((Be, 8), lambda b: (b, 0)),
            pl.BlockSpec((8, 128), lambda b: (0, 0)),
            pl.BlockSpec((128, 128), lambda b: (0, 0)),
        ],
        out_specs=[
            pl.BlockSpec((Be, 128), lambda b: (b, 0)),
            pl.BlockSpec((Be, 128), lambda b: (b, 0)),
        ],
        out_shape=[
            jax.ShapeDtypeStruct((Ep, 128), jnp.float32),
            jax.ShapeDtypeStruct((Ep, 128), jnp.float32),
        ],
        interpret=_INTERPRET,
    )(e, rbf, wrl8, we)


def _edge_msg2_body(e_ref, rbf_ref, esub_ref, wrl_ref, u2_ref):
    rw = _dot(rbf_ref[...], wrl_ref[...])
    u2_ref[...] = _sw(e_ref[...] * rw) * esub_ref[...]


def _edge_msg2(e, rbf, esub, wrl8):
    Ep = e.shape[0]
    Be = _BE
    return pl.pallas_call(
        _edge_msg2_body,
        grid=(Ep // Be,),
        in_specs=[
            pl.BlockSpec((Be, 128), lambda b: (b, 0)),
            pl.BlockSpec((Be, 8), lambda b: (b, 0)),
            pl.BlockSpec((Be, 1), lambda b: (b, 0)),
            pl.BlockSpec((8, 128), lambda b: (0, 0)),
        ],
        out_specs=pl.BlockSpec((Be, 128), lambda b: (b, 0)),
        out_shape=jax.ShapeDtypeStruct((Ep, 128), jnp.float32),
        interpret=_INTERPRET,
    )(e, rbf, esub, wrl8)


def _attn_body(q_ref, kt_ref, mask_ref, vals_ref, idx_ref, *, Bm, Bn, Np):
    pid = pl.program_id(0)
    q = q_ref[...]
    row_ids = pid * Bm + jax.lax.broadcasted_iota(jnp.int32, (Bm, Bn), 0)
    cvals, cidx = [], []
    for cb in range(Np // Bn):
        kt = kt_ref[:, pl.ds(cb * Bn, Bn)]
        s = _dot(q, kt, precision=jax.lax.Precision.HIGHEST) * LAMBDA_ATTN
        col_ids = cb * Bn + jax.lax.broadcasted_iota(jnp.int32, (Bm, Bn), 1)
        s = jnp.where(col_ids == row_ids, s - 1e9, s)
        mb = mask_ref[:, pl.ds(cb * Bn, Bn)]
        s = jnp.where(mb > 0.0, s, -1e30)
        for _ in range(TOPK):
            mx = jnp.max(s, axis=1, keepdims=True)
            eq = s == mx
            am = jnp.min(jnp.where(eq, col_ids, BIG_I32), axis=1, keepdims=True)
            cvals.append(mx)
            cidx.append(am)
            s = jnp.where(col_ids == am, NEG_HUGE, s)
    V = jnp.concatenate(cvals, axis=1)
    I = jnp.concatenate(cidx, axis=1)
    ovals, oidx = [], []
    for _ in range(TOPK):
        mx = jnp.max(V, axis=1, keepdims=True)
        eq = V == mx
        chosen = jnp.min(jnp.where(eq, I, BIG_I32), axis=1, keepdims=True)
        ovals.append(mx)
        oidx.append(chosen)
        V = jnp.where(I == chosen, NEG_HUGE, V)
    vals_ref[...] = jnp.concatenate(ovals, axis=1)
    idx_ref[...] = jnp.concatenate(oidx, axis=1)


def _attention_topk(q_all, k_all, maskf):
    N = q_all.shape[0]
    Np = _ceil_to(N, 2048) if N > 256 else _ceil_to(N, 128)
    Bm = min(256, Np)
    Bn = min(2048, Np)
    qp = _pad_rows(q_all, Np)
    ktp = _pad_rows(k_all, Np).T
    mp = _pad_rows(maskf[:, None], Np).T
    body = functools.partial(_attn_body, Bm=Bm, Bn=Bn, Np=Np)
    vals, idx = pl.pallas_call(
        body,
        grid=(Np // Bm,),
        in_specs=[
            pl.BlockSpec((Bm, S_DIM), lambda b: (b, 0)),
            pl.BlockSpec((S_DIM, Np), lambda b: (0, 0)),
            pl.BlockSpec((1, Np), lambda b: (0, 0)),
        ],
        out_specs=[
            pl.BlockSpec((Bm, TOPK), lambda b: (b, 0)),
            pl.BlockSpec((Bm, TOPK), lambda b: (b, 0)),
        ],
        out_shape=[
            jax.ShapeDtypeStruct((Np, TOPK), jnp.float32),
            jax.ShapeDtypeStruct((Np, TOPK), jnp.int32),
        ],
        interpret=_INTERPRET,
    )(qp, ktp, mp)
    return vals[:N], idx[:N]


def _virt_body(vmsj_ref, vmd_ref, posj_ref, posi_ref, a_ref, valid_ref,
               wrbf_ref, wrl_ref, uv_ref, *, Bd):
    Bn = Bd // TOPK
    vmdi = jnp.broadcast_to(vmd_ref[...][:, None, :], (Bn, TOPK, 128))
    vmdi = vmdi.reshape(Bd, 128)
    posi = jnp.broadcast_to(posi_ref[...][:, None, :], (Bn, TOPK, 128))
    posi = posi.reshape(Bd, 128)
    d = posi - posj_ref[...]
    dist = jnp.sqrt(jnp.sum(d * d, axis=1, keepdims=True) + 1e-9)
    rbf = _rbf8(dist)
    ev = a_ref[...] * _sw(vmsj_ref[...] + vmdi + _dot(rbf, wrbf_ref[...]))
    uv_ref[...] = _sw(ev * _dot(rbf, wrl_ref[...])) * valid_ref[...]


def _virt_msg(vmsj, vmd_p, posj, pos16_p, a, valid, wrbf8, wrl8):
    Mp = vmsj.shape[0]
    Bd = _BE
    body = functools.partial(_virt_body, Bd=Bd)
    return pl.pallas_call(
        body,
        grid=(Mp // Bd,),
        in_specs=[
            pl.BlockSpec((Bd, 128), lambda b: (b, 0)),
            pl.BlockSpec((Bd // TOPK, 128), lambda b: (b, 0)),
            pl.BlockSpec((Bd, 128), lambda b: (b, 0)),
            pl.BlockSpec((Bd // TOPK, 128), lambda b: (b, 0)),
            pl.BlockSpec((Bd, 1), lambda b: (b, 0)),
            pl.BlockSpec((Bd, 1), lambda b: (b, 0)),
            pl.BlockSpec((8, 128), lambda b: (0, 0)),
            pl.BlockSpec((8, 128), lambda b: (0, 0)),
        ],
        out_specs=pl.BlockSpec((Bd, 128), lambda b: (b, 0)),
        out_shape=jax.ShapeDtypeStruct((Mp, 128), jnp.float32),
        interpret=_INTERPRET,
    )(vmsj, vmd_p, posj, pos16_p, a, valid, wrbf8, wrl8)


_SEGSUM = _segsum_sc
_GATHER = _gather_sc


def kernel(z, pos, edge_index, batch, params):
    p = params
    f32 = jnp.float32
    N = pos.shape[0]
    E = edge_index.shape[1]
    Ep = _ceil_to(E, _EALIGN)
    j = edge_index[0]
    i = edge_index[1]
    jp_ = jnp.pad(j, (0, Ep - E)).astype(jnp.int32)
    ip_ = jnp.pad(i, (0, Ep - E)).astype(jnp.int32)
    j2d = jp_.reshape(Ep // _CH, 8, 128)
    i2d = ip_.reshape(Ep // _CH, 8, 128)

    h0 = p["emb_table"][z]
    a_src = h0 @ p["W_src"]
    a_dst = h0 @ p["W_dst"]
    asj = _GATHER(a_src, j2d)
    adi = _GATHER(a_dst, i2d)
    pos128 = jnp.zeros((N, 128), f32).at[:, :3].set(pos)
    posj = _GATHER(pos128, j2d)
    posi = _GATHER(pos128, i2d)

    wrbf8 = jnp.zeros((8, 128), f32).at[:NUM_RADIAL].set(p["W_rbf"])
    e, rbf, u0 = _edge_init(asj, adi, posi, posj, wrbf8, p["W_init_v"], E)
    v = _SEGSUM(u0, i2d, N)

    M = N * TOPK
    Mp = _ceil_to(M, _EALIGN)
    pos128_p = _pad_rows(pos128, Mp // TOPK)
    padvalid = (jnp.arange(Ep, dtype=jnp.int32) < E).astype(f32)[:, None]

    for lp in p["layers"]:
        wrl8 = jnp.zeros((8, 128), f32).at[:NUM_RADIAL].set(lp["W_rbf_l"])
        u, e = _edge_msg(e, rbf, wrl8, lp["W_e"], E)
        v_update = _SEGSUM(u, i2d, N) @ lp["W_msg"]
        v_local = v + v_update
        hs = v_local[:, :S_DIM]
        m = jax.nn.sigmoid(_sw(hs @ lp["W_ms1"] + lp["b_ms1"]) @ lp["W_ms2"]
                           + lp["b_ms2"])[:, 0]
        mask = m > 0.5
        maskf = mask.astype(f32)
        num_master = jnp.sum(mask.astype(jnp.int32))
        esubf = maskf[jp_][:, None] * padvalid

        q_all = hs @ lp["W_q"]
        k_all = hs @ lp["W_k"]
        vals, nbr = _attention_topk(q_all, k_all, maskf)

        kk = jnp.minimum(TOPK, num_master - 1)
        col_valid = jnp.arange(TOPK) < kk
        validf = _pad_rows((mask[:, None] & col_valid[None, :]).astype(f32).reshape(-1, 1), Mp)
        A = _pad_rows(jax.nn.sigmoid(vals).reshape(-1, 1), Mp)
        j_virt = nbr.reshape(-1)
        jv_ = jnp.pad(j_virt, (0, Mp - M)).astype(jnp.int32)
        jv2d = jv_.reshape(Mp // _CH, 8, 128)

        vm_s = v_local @ p["W_src"]
        vm_d = v_local @ p["W_dst"]
        vmsj = _GATHER(vm_s, jv2d)
        posjv = _GATHER(pos128, jv2d)
        vmd_p = _pad_rows(vm_d, Mp // TOPK)
        uv = _virt_msg(vmsj, vmd_p, posjv, pos128_p, A, validf, wrbf8, wrl8)
        s_virt = uv.reshape(Mp // TOPK, TOPK, 128).sum(axis=1)[:N]

        u2 = _edge_msg2(e, rbf, esubf, wrl8)
        s2 = _SEGSUM(u2, i2d, N)
        v_upd_m = (s2 + s_virt) @ lp["W_msg"]

        v_hier = jnp.where(mask[:, None], v_local + v_upd_m, 0.0)
        v_mixed = (1.0 - m[:, None]) * v_local + m[:, None] * v_hier
        v = jnp.where(num_master > 1, v_mixed, v_local)

    pooled = jnp.sum(v, axis=0, keepdims=True)
    out = jax.nn.relu(pooled @ p["W_p1"] + p["b_p1"]) @ p["W_p2"] + p["b_p2"]
    return out
